# baseline (speedup 1.0000x reference)
import functools

import jax
import jax.numpy as jnp
from jax import lax
from jax.experimental import pallas as pl
from jax.experimental.pallas import tpu as pltpu
from jax.experimental.pallas import tpu_sc as plsc

_BATCH = 16384
_DIM = 64

_info = plsc.get_sparse_core_info()
_NC, _NS, _L = _info.num_cores, _info.num_subcores, _info.num_lanes
_NW = _NC * _NS
_BPW = _BATCH // _NW
_CHUNK = 128
_NCH = _BPW // _CHUNK


def _make_sc_kernel():
    mesh = plsc.VectorSubcoreMesh(core_axis_name="c", subcore_axis_name="s")

    @functools.partial(
        pl.kernel,
        mesh=mesh,
        out_type=jax.ShapeDtypeStruct((_BATCH,), jnp.float32),
        compiler_params=pltpu.CompilerParams(
            use_tc_tiling_on_sc=False, needs_layout_passes=False),
        scratch_types=[
            pltpu.VMEM((_BPW,), jnp.int32),
            pltpu.VMEM((_BPW,), jnp.int32),
            pltpu.VMEM((_BPW,), jnp.int32),
            pltpu.VMEM((_BPW, _DIM), jnp.float32),
            pltpu.VMEM((_BPW, _DIM), jnp.float32),
            pltpu.VMEM((_BPW, _DIM), jnp.float32),
            pltpu.VMEM((_BPW,), jnp.float32),
            pltpu.SemaphoreType.DMA,
        ],
    )
    def k(user_hbm, item1_hbm, item2_hbm, utab_hbm, itab_hbm, out_hbm,
          uidx, aidx, bidx, urow, arow, brow, outv, sem):
        wid = lax.axis_index("s") * _NC + lax.axis_index("c")
        base = wid * _BPW
        pltpu.sync_copy(user_hbm.at[pl.ds(base, _BPW)], uidx)
        pltpu.sync_copy(item1_hbm.at[pl.ds(base, _BPW)], aidx)
        pltpu.sync_copy(item2_hbm.at[pl.ds(base, _BPW)], bidx)

        copies = []
        for c in range(_NCH):
            s = pl.ds(c * _CHUNK, _CHUNK)
            copies.append(pltpu.async_copy(utab_hbm.at[uidx.at[s]], urow.at[s], sem))
            copies.append(pltpu.async_copy(itab_hbm.at[aidx.at[s]], arow.at[s], sem))
            copies.append(pltpu.async_copy(itab_hbm.at[bidx.at[s]], brow.at[s], sem))
        for cp in copies:
            cp.wait()

        lanes = lax.iota(jnp.int32, _L)

        def group(g, carry):
            rows = lanes + g * _L
            accs = [jnp.zeros((_L,), jnp.float32) for _ in range(4)]
            for d in range(_DIM):
                col = jnp.full((_L,), d, jnp.int32)
                u = plsc.load_gather(urow, [rows, col])
                a = plsc.load_gather(arow, [rows, col])
                b = plsc.load_gather(brow, [rows, col])
                accs[d % 4] = accs[d % 4] + (u + a) * (u + b)
            dot = (accs[0] + accs[1]) + (accs[2] + accs[3])
            outv[pl.ds(g * _L, _L)] = 1.0 / (1.0 + jnp.exp(-dot))
            return carry

        lax.fori_loop(0, _BPW // _L, group, 0)
        pltpu.sync_copy(outv, out_hbm.at[pl.ds(base, _BPW)])

    return k


_sc_kernel = _make_sc_kernel()


def kernel(user, item1, item2, user_table, item_table):
    return _sc_kernel(user, item1, item2, user_table, item_table)
